# baseline (speedup 1.0000x reference)
import functools

import jax
import jax.numpy as jnp
import numpy as np
from jax import lax
from jax.experimental import pallas as pl
from jax.experimental.pallas import tpu as pltpu
from jax.experimental.pallas import tpu_sc as plsc

OH = 7
OW = 7
SCALE = 0.125
NOR = 8
NCELL = OH * OW
NROW = 8


_YT = 8


def _vgather(a, idx):
    return lax.gather(
        a, idx[:, None],
        lax.GatherDimensionNumbers(
            offset_dims=(), collapsed_slice_dims=(0,), start_index_map=(0,)),
        slice_sizes=(1,),
        mode=lax.GatherScatterMode.PROMISE_IN_BOUNDS)


def _interleave(x):
    C, W = x.shape
    return x.reshape(C // 32, 2, 16, W).swapaxes(1, 2).reshape(C, W)


def _pack_body(W, nyb, f1_ref, f2_ref, o_ref):
    C = f1_ref.shape[1]
    last = pl.program_id(1) == nyb - 1
    for i in range(_YT):
        lo = f1_ref[0, :, i, :]
        if i < _YT - 1:
            hi = f1_ref[0, :, i + 1, :]
        else:
            hi = jnp.where(last, lo, f2_ref[0, :, 0, :])
        row_lo = pltpu.bitcast(
            _interleave(lo).astype(jnp.bfloat16), jnp.int32).T
        row_hi = pltpu.bitcast(
            _interleave(hi).astype(jnp.bfloat16), jnp.int32).T
        o_ref[pl.ds(i * W, W), 0:C // 2] = row_lo
        o_ref[pl.ds(i * W, W), C // 2:C] = row_hi


def _feat_rows(features):
    B, C, H, W = features.shape
    nyb = H // _YT
    return pl.pallas_call(
        functools.partial(_pack_body, W, nyb),
        grid=(B, nyb),
        in_specs=[
            pl.BlockSpec((1, C, _YT, W), lambda b, y: (b, 0, y, 0)),
            pl.BlockSpec((1, C, _YT, W),
                         lambda b, y: (b, 0, jnp.minimum(y + 1, nyb - 1), 0)),
        ],
        out_specs=pl.BlockSpec((_YT * W, C), lambda b, y: (b * nyb + y, 0)),
        out_shape=jax.ShapeDtypeStruct((B * H * W, C), jnp.int32),
    )(features, features)


def _params_body(hw_scalar, r_ref, p_ref):
    r = r_ref[...]
    b = r[0:1, :]
    cx = r[1:2, :] * SCALE
    cy = r[2:3, :] * SCALE
    rw = jnp.maximum(r[3:4, :] * SCALE, 1.0)
    rh = jnp.maximum(r[4:5, :] * SCALE, 1.0)
    th = r[5:6, :]
    indf = th * (NOR / (2.0 * np.pi))
    indfl = jnp.floor(indf)
    l_var = indf - indfl
    ind = jnp.mod(indfl, float(NOR))
    n = r.shape[1]
    p_ref[0:1, :] = jnp.cos(th)
    p_ref[1:2, :] = jnp.sin(th)
    p_ref[2:3, :] = cx
    p_ref[3:4, :] = cy
    p_ref[4:5, :] = rh * 0.5
    p_ref[5:6, :] = rw * 0.5
    p_ref[6:7, :] = rh * (1.0 / OH)
    p_ref[7:8, :] = rw * (1.0 / OW)
    p_ref[8:9, :] = b * hw_scalar
    p_ref[9:10, :] = ind
    p_ref[10:11, :] = l_var
    p_ref[11:12, :] = 1.0 - l_var
    p_ref[12:16, :] = jnp.zeros((4, n), jnp.float32)


def _roi_params(rois, hw):
    n = rois.shape[0]
    p = pl.pallas_call(
        functools.partial(_params_body, float(hw)),
        out_shape=jax.ShapeDtypeStruct((16, n), jnp.float32),
    )(rois.T)
    return p.T


def _sc_body(H, W, C, rois_per_w, nc,
             feat_hbm, params_hbm, out_hbm,
             params_v, idx_buf, w_buf, rows_v, obuf, sems):
    wid = lax.axis_index("s") * nc + lax.axis_index("c")
    base = wid * rois_per_w
    pltpu.sync_copy(params_hbm.at[pl.ds(base, rois_per_w)], params_v)

    lane = lax.iota(jnp.int32, 16)
    r_l = lane & 7
    s_y = lane >> 3
    iy = (r_l >> 2) & 1
    ix = (r_l >> 1) & 1
    xside = r_l & 1
    iy_h = iy.astype(jnp.float32) + 0.5
    ix_h = ix.astype(jnp.float32) + 0.5
    x_hi = xside == 1
    y_hi = s_y == 1
    lane_g8 = (lane >> 3) << 3
    lane_o = lane & 7
    nkv = C // 16

    def roi_body(j, carry):
        prow = params_v[j, :]
        cos_t = prow[0]
        sin_t = prow[1]
        cw = prow[2]
        ch = prow[3]
        hh = prow[4]
        hw2 = prow[5]
        bh = prow[6]
        bw = prow[7]
        boff = prow[8].astype(jnp.int32)
        ind_i = prow[9].astype(jnp.int32)
        l_v = prow[10]
        r_v = prow[11]
        y_off = iy_h * (bh * 0.5) - hh
        x_off = ix_h * (bw * 0.5) - hw2
        rot_lo = lane_g8 + ((lane_o - ind_i + 8) & 7)
        rot_hi = lane_g8 + ((lane_o - ind_i + 9) & 7)

        def idx_body(c, carry2):
            ph = (c // OW).astype(jnp.float32)
            pw = (c % OW).astype(jnp.float32)
            yy = ph * bh + y_off
            xx = pw * bw + x_off
            xr = xx * cos_t - yy * sin_t + cw
            yr = xx * sin_t + yy * cos_t + ch
            valid = ((yr > -1.0) & (yr < float(H))
                     & (xr > -1.0) & (xr < float(W)))
            y0 = jnp.maximum(yr, 0.0)
            x0 = jnp.maximum(xr, 0.0)
            yl0 = y0.astype(jnp.int32)
            xl0 = x0.astype(jnp.int32)
            yl = jnp.minimum(yl0, H - 1)
            xl = jnp.minimum(xl0, W - 1)
            xh = jnp.minimum(xl0 + 1, W - 1)
            yc = jnp.minimum(y0, float(H - 1))
            xc = jnp.minimum(x0, float(W - 1))
            ly = yc - yl.astype(jnp.float32)
            lx = xc - xl.astype(jnp.float32)
            wy = jnp.where(y_hi, ly, 1.0 - ly)
            wx = jnp.where(x_hi, lx, 1.0 - lx)
            w = jnp.where(valid, wy * wx, 0.0) * 0.25
            idx = boff + yl * W + jnp.where(x_hi, xh, xl)
            idx_buf[c // OW, pl.ds((c % OW) * NROW, 16)] = idx
            w_buf[c, :] = w
            return carry2

        def fire(g, carry2):
            pltpu.async_copy(
                feat_hbm.at[idx_buf.at[g, pl.ds(0, OW * NROW)]],
                rows_v.at[g], sems.at[g])
            return carry2

        def acc_chunk(g, carry2):
            pltpu.make_async_copy(
                feat_hbm.at[idx_buf.at[g, pl.ds(0, OW * NROW)]],
                rows_v.at[g], sems.at[g]).wait()
            for cc in range(OW):
                c = g * OW + cc
                rbase = cc * NROW
                w_vec = w_buf[c, :]
                accs = [None] * nkv
                for r in range(NROW):
                    wlo = _vgather(w_vec, jnp.full((16,), r, jnp.int32))
                    whi = _vgather(w_vec, jnp.full((16,), 8 + r, jnp.int32))
                    for half, wv in ((0, wlo), (1, whi)):
                        hbase = half * (C // 2)
                        for jv in range(nkv // 2):
                            rv = plsc.bitcast(
                                rows_v[g, rbase + r,
                                       pl.ds(hbase + jv * 16, 16)],
                                jnp.bfloat16)
                            a, b = plsc.unpack(
                                rv, format=plsc.PackFormat.INTERLEAVED)
                            if accs[2 * jv] is None:
                                accs[2 * jv] = a * wv
                                accs[2 * jv + 1] = b * wv
                            else:
                                accs[2 * jv] = accs[2 * jv] + a * wv
                                accs[2 * jv + 1] = accs[2 * jv + 1] + b * wv
                cell_vec = jnp.full((16,), 0, jnp.int32) + c
                for k in range(nkv):
                    lo = _vgather(accs[k], rot_lo)
                    hi = _vgather(accs[k], rot_hi)
                    ov = r_v * lo + l_v * hi
                    plsc.store_scatter(obuf, [lane + k * 16, cell_vec], ov)
            return carry2

        lax.fori_loop(0, NCELL, idx_body, 0)
        lax.fori_loop(0, OH, fire, 0)
        lax.fori_loop(0, OH, acc_chunk, 0)
        pltpu.sync_copy(obuf, out_hbm.at[base + j])
        return carry

    lax.fori_loop(0, rois_per_w, roi_body, 0)


def _sc_main(feat2d, params, C, H, W):
    n, _ = params.shape
    mesh = plsc.VectorSubcoreMesh(
        core_axis_name="c", subcore_axis_name="s",
        num_cores=2, num_subcores=16)
    nw = mesh.num_cores * mesh.num_subcores
    rois_per_w = n // nw
    body = functools.partial(_sc_body, H, W, C, rois_per_w, mesh.num_cores)
    kern = pl.kernel(
        body,
        out_type=jax.ShapeDtypeStruct((n, C, NCELL), jnp.float32),
        mesh=mesh,
        scratch_types=[
            pltpu.VMEM((rois_per_w, 16), jnp.float32),
            pltpu.VMEM((OH, OW * NROW + 8), jnp.int32),
            pltpu.VMEM((NCELL, 16), jnp.float32),
            pltpu.VMEM((OH, OW * NROW, C), jnp.int32),
            pltpu.VMEM((C, NCELL), jnp.float32),
            pltpu.SemaphoreType.DMA((OH,)),
        ],
        compiler_params=pltpu.CompilerParams(needs_layout_passes=False),
    )
    return kern(feat2d, params)


def kernel(features, rois):
    B, C, H, W = features.shape
    n = rois.shape[0]
    feat2d = _feat_rows(features)
    params = _roi_params(rois, H * W)
    out3 = _sc_main(feat2d, params, C, H, W)
    return out3.reshape(n, C, OH, OW)


# speedup vs baseline: 1.0347x; 1.0347x over previous
import functools

import jax
import jax.numpy as jnp
import numpy as np
from jax import lax
from jax.experimental import pallas as pl
from jax.experimental.pallas import tpu as pltpu
from jax.experimental.pallas import tpu_sc as plsc

OH = 7
OW = 7
SCALE = 0.125
NOR = 8
NCELL = OH * OW
NROW = 8


_YT = 8


def _vgather(a, idx):
    return lax.gather(
        a, idx[:, None],
        lax.GatherDimensionNumbers(
            offset_dims=(), collapsed_slice_dims=(0,), start_index_map=(0,)),
        slice_sizes=(1,),
        mode=lax.GatherScatterMode.PROMISE_IN_BOUNDS)


def _interleave(x):
    C, W = x.shape
    return x.reshape(C // 32, 2, 16, W).swapaxes(1, 2).reshape(C, W)


def _pack_body(W, nyb, f1_ref, f2_ref, o_ref):
    C = f1_ref.shape[1]
    last = pl.program_id(1) == nyb - 1
    for i in range(_YT):
        lo = f1_ref[0, :, i, :]
        if i < _YT - 1:
            hi = f1_ref[0, :, i + 1, :]
        else:
            hi = jnp.where(last, lo, f2_ref[0, :, 0, :])
        row_lo = pltpu.bitcast(
            _interleave(lo).astype(jnp.bfloat16), jnp.int32).T
        row_hi = pltpu.bitcast(
            _interleave(hi).astype(jnp.bfloat16), jnp.int32).T
        o_ref[pl.ds(i * W, W), 0:C // 2] = row_lo
        o_ref[pl.ds(i * W, W), C // 2:C] = row_hi


def _feat_rows(features):
    B, C, H, W = features.shape
    nyb = H // _YT
    return pl.pallas_call(
        functools.partial(_pack_body, W, nyb),
        grid=(B, nyb),
        in_specs=[
            pl.BlockSpec((1, C, _YT, W), lambda b, y: (b, 0, y, 0)),
            pl.BlockSpec((1, C, _YT, W),
                         lambda b, y: (b, 0, jnp.minimum(y + 1, nyb - 1), 0)),
        ],
        out_specs=pl.BlockSpec((_YT * W, C), lambda b, y: (b * nyb + y, 0)),
        out_shape=jax.ShapeDtypeStruct((B * H * W, C), jnp.int32),
    )(features, features)


def _params_body(hw_scalar, r_ref, p_ref):
    r = r_ref[...]
    b = r[0:1, :]
    cx = r[1:2, :] * SCALE
    cy = r[2:3, :] * SCALE
    rw = jnp.maximum(r[3:4, :] * SCALE, 1.0)
    rh = jnp.maximum(r[4:5, :] * SCALE, 1.0)
    th = r[5:6, :]
    indf = th * (NOR / (2.0 * np.pi))
    indfl = jnp.floor(indf)
    l_var = indf - indfl
    ind = jnp.mod(indfl, float(NOR))
    n = r.shape[1]
    p_ref[0:1, :] = jnp.cos(th)
    p_ref[1:2, :] = jnp.sin(th)
    p_ref[2:3, :] = cx
    p_ref[3:4, :] = cy
    p_ref[4:5, :] = rh * 0.5
    p_ref[5:6, :] = rw * 0.5
    p_ref[6:7, :] = rh * (1.0 / OH)
    p_ref[7:8, :] = rw * (1.0 / OW)
    p_ref[8:9, :] = b * hw_scalar
    p_ref[9:10, :] = ind
    p_ref[10:11, :] = l_var
    p_ref[11:12, :] = 1.0 - l_var
    p_ref[12:16, :] = jnp.zeros((4, n), jnp.float32)


def _roi_params(rois, hw):
    n = rois.shape[0]
    p = pl.pallas_call(
        functools.partial(_params_body, float(hw)),
        out_shape=jax.ShapeDtypeStruct((16, n), jnp.float32),
    )(rois.T)
    return p.T


def _sc_body(H, W, C, rois_per_w, nc,
             feat_hbm, params_hbm, out_hbm,
             params_v, idx_buf, w_buf, rows_v, obuf, sems, semo):
    wid = lax.axis_index("s") * nc + lax.axis_index("c")
    base = wid * rois_per_w
    pltpu.sync_copy(params_hbm.at[pl.ds(base, rois_per_w)], params_v)

    pltpu.async_copy(obuf.at[0], out_hbm.at[base], semo.at[0])
    pltpu.async_copy(obuf.at[1], out_hbm.at[base + 1], semo.at[1])

    lane = lax.iota(jnp.int32, 16)
    r_l = lane & 7
    s_y = lane >> 3
    iy = (r_l >> 2) & 1
    ix = (r_l >> 1) & 1
    xside = r_l & 1
    iy_h = iy.astype(jnp.float32) + 0.5
    ix_h = ix.astype(jnp.float32) + 0.5
    x_hi = xside == 1
    y_hi = s_y == 1
    lane_g8 = (lane >> 3) << 3
    lane_o = lane & 7
    nkv = C // 16

    def roi_body(j, carry):
        slot = j & 1
        prow = params_v[j, :]
        cos_t = prow[0]
        sin_t = prow[1]
        cw = prow[2]
        ch = prow[3]
        hh = prow[4]
        hw2 = prow[5]
        bh = prow[6]
        bw = prow[7]
        boff = prow[8].astype(jnp.int32)
        ind_i = prow[9].astype(jnp.int32)
        l_v = prow[10]
        r_v = prow[11]
        y_off = iy_h * (bh * 0.5) - hh
        x_off = ix_h * (bw * 0.5) - hw2
        rot_lo = lane_g8 + ((lane_o - ind_i + 8) & 7)
        rot_hi = lane_g8 + ((lane_o - ind_i + 9) & 7)

        def idx_body(c, carry2):
            ph = (c // OW).astype(jnp.float32)
            pw = (c % OW).astype(jnp.float32)
            yy = ph * bh + y_off
            xx = pw * bw + x_off
            xr = xx * cos_t - yy * sin_t + cw
            yr = xx * sin_t + yy * cos_t + ch
            valid = ((yr > -1.0) & (yr < float(H))
                     & (xr > -1.0) & (xr < float(W)))
            y0 = jnp.maximum(yr, 0.0)
            x0 = jnp.maximum(xr, 0.0)
            yl0 = y0.astype(jnp.int32)
            xl0 = x0.astype(jnp.int32)
            yl = jnp.minimum(yl0, H - 1)
            xl = jnp.minimum(xl0, W - 1)
            xh = jnp.minimum(xl0 + 1, W - 1)
            yc = jnp.minimum(y0, float(H - 1))
            xc = jnp.minimum(x0, float(W - 1))
            ly = yc - yl.astype(jnp.float32)
            lx = xc - xl.astype(jnp.float32)
            wy = jnp.where(y_hi, ly, 1.0 - ly)
            wx = jnp.where(x_hi, lx, 1.0 - lx)
            w = jnp.where(valid, wy * wx, 0.0) * 0.25
            idx = boff + yl * W + jnp.where(x_hi, xh, xl)
            idx_buf[c // OW, pl.ds((c % OW) * NROW, 16)] = idx
            w_buf[c, :] = w
            return carry2

        def fire(g, carry2):
            pltpu.async_copy(
                feat_hbm.at[idx_buf.at[g, pl.ds(0, OW * NROW)]],
                rows_v.at[g], sems.at[g])
            return carry2

        def acc_chunk(g, carry2):
            pltpu.make_async_copy(
                feat_hbm.at[idx_buf.at[g, pl.ds(0, OW * NROW)]],
                rows_v.at[g], sems.at[g]).wait()
            for cc in range(OW):
                c = g * OW + cc
                rbase = cc * NROW
                w_vec = w_buf[c, :]
                accs = [None] * nkv
                for r in range(NROW):
                    wlo = _vgather(w_vec, jnp.full((16,), r, jnp.int32))
                    whi = _vgather(w_vec, jnp.full((16,), 8 + r, jnp.int32))
                    for half, wv in ((0, wlo), (1, whi)):
                        hbase = half * (C // 2)
                        for jv in range(nkv // 2):
                            rv = plsc.bitcast(
                                rows_v[g, rbase + r,
                                       pl.ds(hbase + jv * 16, 16)],
                                jnp.bfloat16)
                            a, b = plsc.unpack(
                                rv, format=plsc.PackFormat.INTERLEAVED)
                            if accs[2 * jv] is None:
                                accs[2 * jv] = a * wv
                                accs[2 * jv + 1] = b * wv
                            else:
                                accs[2 * jv] = accs[2 * jv] + a * wv
                                accs[2 * jv + 1] = accs[2 * jv + 1] + b * wv
                cell_vec = jnp.full((16,), 0, jnp.int32) + c
                for k in range(nkv):
                    lo = _vgather(accs[k], rot_lo)
                    hi = _vgather(accs[k], rot_hi)
                    ov = r_v * lo + l_v * hi
                    plsc.store_scatter(obuf.at[slot],
                                       [lane + k * 16, cell_vec], ov)
            return carry2

        lax.fori_loop(0, NCELL, idx_body, 0)
        lax.fori_loop(0, OH, fire, 0)
        pltpu.make_async_copy(
            obuf.at[slot], out_hbm.at[base + j], semo.at[slot]).wait()
        lax.fori_loop(0, OH, acc_chunk, 0)
        pltpu.async_copy(obuf.at[slot], out_hbm.at[base + j], semo.at[slot])
        return carry

    lax.fori_loop(0, rois_per_w, roi_body, 0)
    for s in range(2):
        jlast = rois_per_w - 2 + s
        pltpu.make_async_copy(
            obuf.at[jlast & 1], out_hbm.at[base + jlast],
            semo.at[jlast & 1]).wait()


def _sc_main(feat2d, params, C, H, W):
    n, _ = params.shape
    mesh = plsc.VectorSubcoreMesh(
        core_axis_name="c", subcore_axis_name="s",
        num_cores=2, num_subcores=16)
    nw = mesh.num_cores * mesh.num_subcores
    rois_per_w = n // nw
    body = functools.partial(_sc_body, H, W, C, rois_per_w, mesh.num_cores)
    kern = pl.kernel(
        body,
        out_type=jax.ShapeDtypeStruct((n, C, NCELL), jnp.float32),
        mesh=mesh,
        scratch_types=[
            pltpu.VMEM((rois_per_w, 16), jnp.float32),
            pltpu.VMEM((OH, OW * NROW + 8), jnp.int32),
            pltpu.VMEM((NCELL, 16), jnp.float32),
            pltpu.VMEM((OH, OW * NROW, C), jnp.int32),
            pltpu.VMEM((2, C, NCELL), jnp.float32),
            pltpu.SemaphoreType.DMA((OH,)),
            pltpu.SemaphoreType.DMA((2,)),
        ],
        compiler_params=pltpu.CompilerParams(needs_layout_passes=False),
    )
    return kern(feat2d, params)


def kernel(features, rois):
    B, C, H, W = features.shape
    n = rois.shape[0]
    feat2d = _feat_rows(features)
    params = _roi_params(rois, H * W)
    out3 = _sc_main(feat2d, params, C, H, W)
    return out3.reshape(n, C, OH, OW)
